# baseline (speedup 1.0000x reference)
import functools

import jax
import jax.numpy as jnp
from jax import lax
from jax.experimental import pallas as pl
from jax.experimental.pallas import tpu as pltpu
from jax.experimental.pallas import tpu_sc as plsc

NC = 2
NT = 16
LANES = 16
CHUNK = 128
DH = 64
IDX_BLK = 52


def _build_sc_call(n, n_pad, n_chunks):
  acc_rows = 2 * n_pad
  cpt = n_chunks // NT
  nblk = cpt // IDX_BLK
  assert nblk % 2 == 0 and cpt % IDX_BLK == 0 and IDX_BLK % 2 == 0
  rows_per_tile = acc_rows // NT

  mesh = plsc.VectorSubcoreMesh(core_axis_name="c", subcore_axis_name="s")

  @functools.partial(
      pl.kernel,
      mesh=mesh,
      compiler_params=pltpu.CompilerParams(use_tc_tiling_on_sc=False),
      out_type=jax.ShapeDtypeStruct((NC * acc_rows, DH), jnp.float32),
      scratch_types=[
          pltpu.VMEM((2, IDX_BLK, CHUNK), jnp.int32),
          pltpu.VMEM((2, CHUNK), jnp.int32),
          pltpu.VMEM((2, CHUNK), jnp.int32),
          pltpu.VMEM((2, CHUNK, DH), jnp.float32),
          pltpu.VMEM_SHARED((acc_rows, DH), jnp.float32),
          [pltpu.SemaphoreType.DMA] * 2,
          [pltpu.SemaphoreType.DMA] * 2,
      ],
  )
  def sc_kernel(x_hbm, pk_hbm, out_hbm,
                pkbuf, srcb, dstb, rows, acc, isem, gsem):
    cid = lax.axis_index("c")
    tid = lax.axis_index("s")

    def i_desc(bi, h):
      return pltpu.make_async_copy(
          pk_hbm.at[pl.ds(tid * cpt + bi * IDX_BLK, IDX_BLK)], pkbuf.at[h],
          isem[h])

    i_desc(0, 0).start()

    zbuf = rows.at[0]

    def zrow(r, carry):
      for j in range(DH // LANES):
        zbuf[r, pl.ds(j * LANES, LANES)] = jnp.zeros((LANES,), jnp.float32)
      return carry

    lax.fori_loop(0, CHUNK, zrow, 0)

    def zcp(q, carry):
      pltpu.sync_copy(zbuf, acc.at[pl.ds(tid * rows_per_tile + q * CHUNK, CHUNK)])
      return carry

    lax.fori_loop(0, rows_per_tile // CHUNK, zcp, 0)
    plsc.subcore_barrier()

    def unpack(h, kl, b):
      for j in range(CHUNK // LANES):
        pw = pkbuf[h, kl, pl.ds(j * LANES, LANES)]
        srcb[b, pl.ds(j * LANES, LANES)] = ((pw & 0xFFFF) << 1) + cid
        dstb[b, pl.ds(j * LANES, LANES)] = pw >> 16

    def g_desc(b):
      return pltpu.make_async_copy(x_hbm.at[srcb.at[b]], rows.at[b], gsem[b])

    def scatter_add(b):
      pltpu.sync_copy(rows.at[b], acc.at[dstb.at[b]], add=True)

    def process_block(bi, h):
      i_desc(bi, h).wait()

      @pl.when(bi + 1 < nblk)
      def _():
        i_desc(bi + 1, 1 - h).start()

      unpack(h, 0, 0)
      g_desc(0).start()

      def pair_body(p, carry):
        k0 = 2 * p
        unpack(h, k0 + 1, 1)
        g_desc(1).start()
        g_desc(0).wait()
        scatter_add(0)

        @pl.when(k0 + 2 < IDX_BLK)
        def _():
          unpack(h, k0 + 2, 0)
          g_desc(0).start()

        g_desc(1).wait()
        scatter_add(1)
        return carry

      lax.fori_loop(0, IDX_BLK // 2, pair_body, 0)

    def pair(bp, carry):
      process_block(2 * bp, 0)
      process_block(2 * bp + 1, 1)
      return carry

    lax.fori_loop(0, nblk // 2, pair, 0)

    plsc.subcore_barrier()
    out_row0 = cid * acc_rows + tid * rows_per_tile
    pltpu.sync_copy(acc.at[pl.ds(tid * rows_per_tile, rows_per_tile)],
                    out_hbm.at[pl.ds(out_row0, rows_per_tile)])

  return sc_kernel, acc_rows


def kernel(x, edge_index, edge_index2):
  n, d = x.shape
  assert d == 2 * DH
  n_pad = ((n + 1023) // 1024) * 1024
  dummy = n_pad - 1

  x2h = x.reshape(2 * n, DH)

  src = jnp.concatenate([edge_index[1], edge_index2[1]])
  dst = jnp.concatenate([edge_index[0], edge_index2[0] + n_pad])
  e_tot = src.shape[0]
  grain = NT * 2 * IDX_BLK * CHUNK
  e_pad = ((e_tot + grain - 1) // grain) * grain
  src = jnp.pad(src, (0, e_pad - e_tot))
  dst = jnp.pad(dst, (0, e_pad - e_tot), constant_values=dummy)
  packed = (dst << 16) | src
  n_chunks = e_pad // CHUNK
  pk = packed.reshape(n_chunks, CHUNK)

  sc_call, acc_rows = _build_sc_call(n, n_pad, n_chunks)
  o = sc_call(x2h, pk)
  o0, o1 = o[:acc_rows], o[acc_rows:]
  x1 = jnp.concatenate([o0[:n], o1[:n]], axis=1)
  x2 = jnp.concatenate([o0[n_pad:n_pad + n], o1[n_pad:n_pad + n]], axis=1)
  return jnp.concatenate([x1, x2], axis=1)


# speedup vs baseline: 1.4197x; 1.4197x over previous
import functools

import jax
import jax.numpy as jnp
from jax import lax
from jax.experimental import pallas as pl
from jax.experimental.pallas import tpu as pltpu
from jax.experimental.pallas import tpu_sc as plsc

NC = 2
NT = 16
LANES = 16
CHUNK = 128
DH = 64
IDX_BLK = 52


def _build_sc_call(n, n_pad, n_chunks):
  acc_rows = 2 * n_pad
  cpt = n_chunks // NT
  nblk = cpt // IDX_BLK
  assert nblk % 2 == 0 and cpt % IDX_BLK == 0 and IDX_BLK % 2 == 0
  rows_per_tile = acc_rows // NT

  mesh = plsc.VectorSubcoreMesh(core_axis_name="c", subcore_axis_name="s")

  @functools.partial(
      pl.kernel,
      mesh=mesh,
      compiler_params=pltpu.CompilerParams(use_tc_tiling_on_sc=False),
      out_type=jax.ShapeDtypeStruct((NC * acc_rows, DH), jnp.float32),
      scratch_types=[
          pltpu.VMEM((2, IDX_BLK, CHUNK), jnp.int32),
          pltpu.VMEM((2, CHUNK), jnp.int32),
          pltpu.VMEM((2, CHUNK), jnp.int32),
          pltpu.VMEM((2, CHUNK, DH), jnp.float32),
          pltpu.VMEM_SHARED((acc_rows, DH), jnp.float32),
          [pltpu.SemaphoreType.DMA] * 2,
          [pltpu.SemaphoreType.DMA] * 2,
      ],
  )
  def sc_kernel(x_hbm, pk_hbm, out_hbm,
                pkbuf, srcb, dstb, rows, acc, isem, gsem):
    cid = lax.axis_index("c")
    tid = lax.axis_index("s")

    def i_desc(bi, h):
      return pltpu.make_async_copy(
          pk_hbm.at[pl.ds(tid * cpt + bi * IDX_BLK, IDX_BLK)], pkbuf.at[h],
          isem[h])

    i_desc(0, 0).start()

    zbuf = rows.at[0]

    def zrow(r, carry):
      for j in range(DH // LANES):
        zbuf[r, pl.ds(j * LANES, LANES)] = jnp.zeros((LANES,), jnp.float32)
      return carry

    lax.fori_loop(0, CHUNK, zrow, 0)

    def zcp(q, carry):
      pltpu.sync_copy(zbuf, acc.at[pl.ds(tid * rows_per_tile + q * CHUNK, CHUNK)])
      return carry

    lax.fori_loop(0, rows_per_tile // CHUNK, zcp, 0)
    plsc.subcore_barrier()

    row_base = cid * n

    def unpack(h, kl, b):
      for j in range(CHUNK // LANES):
        pw = pkbuf[h, kl, pl.ds(j * LANES, LANES)]
        srcb[b, pl.ds(j * LANES, LANES)] = (pw & 0xFFFF) + row_base
        dstb[b, pl.ds(j * LANES, LANES)] = pw >> 16

    def g_desc(b):
      return pltpu.make_async_copy(x_hbm.at[srcb.at[b]], rows.at[b], gsem[b])

    def scatter_add(b):
      pltpu.sync_copy(rows.at[b], acc.at[dstb.at[b]], add=True)

    def process_block(bi, h):
      i_desc(bi, h).wait()

      @pl.when(bi + 1 < nblk)
      def _():
        i_desc(bi + 1, 1 - h).start()

      unpack(h, 0, 0)
      g_desc(0).start()

      def pair_body(p, carry):
        k0 = 2 * p
        unpack(h, k0 + 1, 1)
        g_desc(1).start()
        g_desc(0).wait()
        scatter_add(0)

        @pl.when(k0 + 2 < IDX_BLK)
        def _():
          unpack(h, k0 + 2, 0)
          g_desc(0).start()

        g_desc(1).wait()
        scatter_add(1)
        return carry

      lax.fori_loop(0, IDX_BLK // 2, pair_body, 0)

    def pair(bp, carry):
      process_block(2 * bp, 0)
      process_block(2 * bp + 1, 1)
      return carry

    lax.fori_loop(0, nblk // 2, pair, 0)

    plsc.subcore_barrier()
    out_row0 = cid * acc_rows + tid * rows_per_tile
    pltpu.sync_copy(acc.at[pl.ds(tid * rows_per_tile, rows_per_tile)],
                    out_hbm.at[pl.ds(out_row0, rows_per_tile)])

  return sc_kernel, acc_rows


def kernel(x, edge_index, edge_index2):
  n, d = x.shape
  assert d == 2 * DH
  n_pad = ((n + 1023) // 1024) * 1024
  dummy = n_pad - 1

  x2h = jnp.concatenate([x[:, :DH], x[:, DH:]], axis=0)

  src = jnp.concatenate([edge_index[1], edge_index2[1]])
  dst = jnp.concatenate([edge_index[0], edge_index2[0] + n_pad])
  e_tot = src.shape[0]
  grain = NT * 2 * IDX_BLK * CHUNK
  e_pad = ((e_tot + grain - 1) // grain) * grain
  src = jnp.pad(src, (0, e_pad - e_tot))
  dst = jnp.pad(dst, (0, e_pad - e_tot), constant_values=dummy)
  packed = (dst << 16) | src
  n_chunks = e_pad // CHUNK
  pk = packed.reshape(n_chunks, CHUNK)

  sc_call, acc_rows = _build_sc_call(n, n_pad, n_chunks)
  o = sc_call(x2h, pk)
  o0, o1 = o[:acc_rows], o[acc_rows:]
  x1 = jnp.concatenate([o0[:n], o1[:n]], axis=1)
  x2 = jnp.concatenate([o0[n_pad:n_pad + n], o1[n_pad:n_pad + n]], axis=1)
  return jnp.concatenate([x1, x2], axis=1)
